# baseline (speedup 1.0000x reference)
import jax
import jax.numpy as jnp
from jax.experimental import pallas as pl
from jax.experimental.pallas import tpu as pltpu

_CP = 128
_K = 5
_H0, _W0 = 28, 28
_OH1, _OW1 = 24, 24
_PH1, _PW1 = 12, 12
_OH2, _OW2 = 8, 8
_PH2, _PW2 = 4, 4
_TB = 32
_N1 = _OW1 * _CP
_KC1 = 8 * _W0
_HALF = _PW1 * _CP


def _fused_kernel(xa_ref, xb_ref, xc_ref, xd_ref, m1_ref, b1_ref, w2_ref,
                  b2_ref, wfc_ref, bfc_ref, o_ref):
    tb = xa_ref.shape[0]
    xq = [xa_ref[...], xb_ref[...], xc_ref[...], xd_ref[...]]

    xk = jnp.concatenate(
        [xq[m][:, 0:6, :] for m in range(4)] +
        [xq[m][:, 1:7, :] for m in range(4)], axis=2)
    xk = xk.reshape(tb * 6, _KC1)
    acc = jnp.dot(xk, m1_ref[...], preferred_element_type=jnp.float32)
    h = jnp.maximum(acc + b1_ref[...], 0.0)

    def gather(dq, par):
        return jnp.concatenate(
            [h[:, (2 * s + dq) * _N1 + (2 * j + par) * _CP:
                 (2 * s + dq) * _N1 + (2 * j + par + 1) * _CP]
             for s in range(2) for j in range(_PW1)], axis=1)
    p1 = jnp.maximum(jnp.maximum(gather(0, 0), gather(0, 1)),
                     jnp.maximum(gather(1, 0), gather(1, 1)))
    p1 = p1.astype(jnp.bfloat16).reshape(tb, 6, 2 * _HALF)

    ru = [p1[:, u:u + _PH2, :] for u in range(3)]
    cols = []
    for j in range(_PW2):
        lj = jnp.concatenate(
            [r[:, :, s * _HALF + 2 * j * _CP:s * _HALF + (2 * j + 6) * _CP]
             for r in ru for s in range(2)], axis=2)
        lj = lj.reshape(tb * _PH2, 6 * 6 * _CP)
        accj = jnp.dot(lj, w2_ref[...], preferred_element_type=jnp.float32)
        accj = jnp.maximum(accj + b2_ref[...], 0.0)
        cols.append(jnp.maximum(
            jnp.maximum(accj[:, 0:_CP], accj[:, _CP:2 * _CP]),
            jnp.maximum(accj[:, 2 * _CP:3 * _CP], accj[:, 3 * _CP:])))
    p2 = jnp.concatenate(cols, axis=1)
    p2 = p2.reshape(tb, _PH2, _PW2 * _CP)

    p2w = jnp.concatenate([p2[:, t, :] for t in range(_PH2)], axis=1)
    p2w = p2w.astype(jnp.bfloat16)
    logits = jnp.dot(p2w, wfc_ref[...],
                     preferred_element_type=jnp.float32) + bfc_ref[...]
    o_ref[...] = logits


def _pack_m1_banded(m1):
    zero = jnp.zeros((_W0, _N1), m1.dtype)
    rows = []
    for r in range(8):
        quarters = []
        for q in range(4):
            kh = r - q
            quarters.append(m1[kh] if 0 <= kh < _K else zero)
        rows.append(jnp.concatenate(quarters, axis=1))
    return jnp.concatenate(rows, axis=0)


def _pack_w2_banded(w2p):
    w2r = w2p.reshape(_K, _K, _CP, _CP)
    zero = jnp.zeros((_CP, _CP), w2p.dtype)
    rows = []
    for u in range(3):
        for s in range(2):
            for wpp in range(6):
                cells = []
                for t in range(2):
                    for opp in range(2):
                        kh = 2 * u + s - t
                        kw = wpp - opp
                        ok = 0 <= kh < _K and 0 <= kw < _K
                        cells.append(w2r[kh, kw] if ok else zero)
                rows.append(jnp.concatenate(cells, axis=1))
    return jnp.concatenate(rows, axis=0)


def kernel(x, m1, b1big, w2p, b2p, wfc_flat, bfcp):
    B = x.shape[0]
    xs = x.reshape(B, _H0, _W0).astype(jnp.bfloat16)
    tb = _TB if B >= _TB else B
    Bp = ((B + tb - 1) // tb) * tb
    if Bp != B:
        xs = jnp.pad(xs, ((0, Bp - B), (0, 0), (0, 0)))
    xsplit = [xs[:, m::4, :] for m in range(4)]

    m1b = _pack_m1_banded(m1).astype(jnp.bfloat16)
    b1w = jnp.concatenate([b1big] * 4, axis=1)
    w2b = _pack_w2_banded(w2p).astype(jnp.bfloat16)
    b2w = jnp.concatenate([b2p] * 4, axis=1)
    wfcb = wfc_flat.astype(jnp.bfloat16)

    grid = (Bp // tb,)
    xspec = pl.BlockSpec((tb, 7, _W0), lambda i: (i, 0, 0))
    out = pl.pallas_call(
        _fused_kernel,
        out_shape=jax.ShapeDtypeStruct((Bp, _CP), jnp.float32),
        grid=grid,
        in_specs=[
            xspec, xspec, xspec, xspec,
            pl.BlockSpec((_KC1, 4 * _N1), lambda i: (0, 0)),
            pl.BlockSpec((1, 4 * _N1), lambda i: (0, 0)),
            pl.BlockSpec((36 * _CP, 4 * _CP), lambda i: (0, 0)),
            pl.BlockSpec((1, 4 * _CP), lambda i: (0, 0)),
            pl.BlockSpec((_PH2 * _PW2 * _CP, _CP), lambda i: (0, 0)),
            pl.BlockSpec((1, _CP), lambda i: (0, 0)),
        ],
        out_specs=pl.BlockSpec((tb, _CP), lambda i: (i, 0)),
        compiler_params=pltpu.CompilerParams(
            dimension_semantics=("parallel",),
            vmem_limit_bytes=64 * 1024 * 1024),
    )(*xsplit, m1b, b1w, w2b, b2w, wfcb, bfcp)

    return out[:B, :10]


# speedup vs baseline: 1.7059x; 1.7059x over previous
import jax
import jax.numpy as jnp
from jax.experimental import pallas as pl
from jax.experimental.pallas import tpu as pltpu

_CP = 128
_K = 5
_H0, _W0 = 28, 28
_OH1, _OW1 = 24, 24
_PH1, _PW1 = 12, 12
_OH2, _OW2 = 8, 8
_PH2, _PW2 = 4, 4
_TB = 32
_N1 = _OW1 * _CP
_KC1 = 8 * _W0
_HALF = _PW1 * _CP


def _fused_kernel(xa_ref, xb_ref, xc_ref, xd_ref, m1_ref, b1_ref, w2_ref,
                  b2_ref, wfc_ref, bfc_ref, o_ref):
    tb = xa_ref.shape[1]
    xq = [xa_ref[...], xb_ref[...], xc_ref[...], xd_ref[...]]

    xk = jnp.concatenate(
        [xq[m][0:6] for m in range(4)] +
        [xq[m][1:7] for m in range(4)], axis=2)
    xk = xk.reshape(6 * tb, _KC1)
    acc = jnp.dot(xk, m1_ref[...], preferred_element_type=jnp.float32)
    h = jnp.maximum(acc + b1_ref[...], 0.0)

    def gather(dq, par):
        return jnp.concatenate(
            [h[:, (2 * s + dq) * _N1 + (2 * j + par) * _CP:
                 (2 * s + dq) * _N1 + (2 * j + par + 1) * _CP]
             for s in range(2) for j in range(_PW1)], axis=1)
    p1 = jnp.maximum(jnp.maximum(gather(0, 0), gather(0, 1)),
                     jnp.maximum(gather(1, 0), gather(1, 1)))
    p1 = p1.astype(jnp.bfloat16)

    ljs = []
    for j in range(_PW2):
        ljs.append(jnp.concatenate(
            [p1[u * tb:(u + 4) * tb,
                s * _HALF + 2 * j * _CP:s * _HALF + (2 * j + 6) * _CP]
             for u in range(3) for s in range(2)], axis=1))
    big = jnp.concatenate(ljs, axis=0)
    acc2 = jnp.dot(big, w2_ref[...], preferred_element_type=jnp.float32)
    acc2 = jnp.maximum(acc2 + b2_ref[...], 0.0)

    pooled = jnp.maximum(
        jnp.maximum(acc2[:, 0:_CP], acc2[:, _CP:2 * _CP]),
        jnp.maximum(acc2[:, 2 * _CP:3 * _CP], acc2[:, 3 * _CP:]))

    p2w = jnp.concatenate(
        [pooled[(j * _PH2 + oh2) * tb:(j * _PH2 + oh2 + 1) * tb, :]
         for oh2 in range(_PH2) for j in range(_PW2)], axis=1)
    logits = jnp.dot(p2w.astype(jnp.bfloat16), wfc_ref[...],
                     preferred_element_type=jnp.float32) + bfc_ref[...]
    o_ref[...] = logits


def _pack_m1_banded(m1):
    zero = jnp.zeros((_W0, _N1), m1.dtype)
    rows = []
    for r in range(8):
        quarters = []
        for q in range(4):
            kh = r - q
            quarters.append(m1[kh] if 0 <= kh < _K else zero)
        rows.append(jnp.concatenate(quarters, axis=1))
    return jnp.concatenate(rows, axis=0)


def _pack_w2_banded(w2p):
    w2r = w2p.reshape(_K, _K, _CP, _CP)
    zero = jnp.zeros((_CP, _CP), w2p.dtype)
    rows = []
    for u in range(3):
        for s in range(2):
            for wpp in range(6):
                cells = []
                for t in range(2):
                    for opp in range(2):
                        kh = 2 * u + s - t
                        kw = wpp - opp
                        ok = 0 <= kh < _K and 0 <= kw < _K
                        cells.append(w2r[kh, kw] if ok else zero)
                rows.append(jnp.concatenate(cells, axis=1))
    return jnp.concatenate(rows, axis=0)


def kernel(x, m1, b1big, w2p, b2p, wfc_flat, bfcp):
    B = x.shape[0]
    xs = x.reshape(B, _H0, _W0).astype(jnp.bfloat16)
    tb = _TB if B >= _TB else B
    Bp = ((B + tb - 1) // tb) * tb
    if Bp != B:
        xs = jnp.pad(xs, ((0, Bp - B), (0, 0), (0, 0)))
    xsplit = [jnp.transpose(xs[:, m::4, :], (1, 0, 2)) for m in range(4)]

    m1b = _pack_m1_banded(m1).astype(jnp.bfloat16)
    b1w = jnp.concatenate([b1big] * 4, axis=1)
    w2b = _pack_w2_banded(w2p).astype(jnp.bfloat16)
    b2w = jnp.concatenate([b2p] * 4, axis=1)
    wfcb = wfc_flat.astype(jnp.bfloat16)

    grid = (Bp // tb,)
    xspec = pl.BlockSpec((7, tb, _W0), lambda i: (0, i, 0))
    out = pl.pallas_call(
        _fused_kernel,
        out_shape=jax.ShapeDtypeStruct((Bp, _CP), jnp.float32),
        grid=grid,
        in_specs=[
            xspec, xspec, xspec, xspec,
            pl.BlockSpec((_KC1, 4 * _N1), lambda i: (0, 0)),
            pl.BlockSpec((1, 4 * _N1), lambda i: (0, 0)),
            pl.BlockSpec((36 * _CP, 4 * _CP), lambda i: (0, 0)),
            pl.BlockSpec((1, 4 * _CP), lambda i: (0, 0)),
            pl.BlockSpec((_PH2 * _PW2 * _CP, _CP), lambda i: (0, 0)),
            pl.BlockSpec((1, _CP), lambda i: (0, 0)),
        ],
        out_specs=pl.BlockSpec((tb, _CP), lambda i: (i, 0)),
        compiler_params=pltpu.CompilerParams(
            dimension_semantics=("parallel",),
            vmem_limit_bytes=64 * 1024 * 1024),
    )(*xsplit, m1b, b1w, w2b, b2w, wfcb, bfcp)

    return out[:B, :10]


# speedup vs baseline: 1.7978x; 1.0539x over previous
import jax
import jax.numpy as jnp
from jax.experimental import pallas as pl
from jax.experimental.pallas import tpu as pltpu

_CP = 128
_K = 5
_H0, _W0 = 28, 28
_OH1, _OW1 = 24, 24
_PH1, _PW1 = 12, 12
_OH2, _OW2 = 8, 8
_PH2, _PW2 = 4, 4
_TB = 64
_N1 = _OW1 * _CP
_KC1 = 8 * _W0
_HALF = _PW1 * _CP


def _fused_kernel(xa_ref, xb_ref, xc_ref, xd_ref, m1_ref, b1_ref, w2_ref,
                  b2_ref, wfc_ref, bfc_ref, o_ref):
    tb = xa_ref.shape[1]
    xq = [xa_ref[...], xb_ref[...], xc_ref[...], xd_ref[...]]

    xk = jnp.concatenate(
        [xq[m][0:6] for m in range(4)] +
        [xq[m][1:7] for m in range(4)], axis=2)
    xk = xk.reshape(6 * tb, _KC1)
    acc = jnp.dot(xk, m1_ref[...], preferred_element_type=jnp.float32)
    h = jnp.maximum(acc + b1_ref[...], 0.0)

    def gather(dq, par):
        return jnp.concatenate(
            [h[:, (2 * s + dq) * _N1 + (2 * j + par) * _CP:
                 (2 * s + dq) * _N1 + (2 * j + par + 1) * _CP]
             for s in range(2) for j in range(_PW1)], axis=1)
    p1 = jnp.maximum(jnp.maximum(gather(0, 0), gather(0, 1)),
                     jnp.maximum(gather(1, 0), gather(1, 1)))
    p1 = p1.astype(jnp.bfloat16)

    ljs = []
    for j in range(_PW2):
        ljs.append(jnp.concatenate(
            [p1[u * tb:(u + 4) * tb,
                s * _HALF + 2 * j * _CP:s * _HALF + (2 * j + 6) * _CP]
             for u in range(3) for s in range(2)], axis=1))
    big = jnp.concatenate(ljs, axis=0)
    acc2 = jnp.dot(big, w2_ref[...], preferred_element_type=jnp.float32)
    acc2 = jnp.maximum(acc2 + b2_ref[...], 0.0)

    pooled = jnp.maximum(
        jnp.maximum(acc2[:, 0:_CP], acc2[:, _CP:2 * _CP]),
        jnp.maximum(acc2[:, 2 * _CP:3 * _CP], acc2[:, 3 * _CP:]))

    p2w = jnp.concatenate(
        [pooled[(j * _PH2 + oh2) * tb:(j * _PH2 + oh2 + 1) * tb, :]
         for oh2 in range(_PH2) for j in range(_PW2)], axis=1)
    logits = jnp.dot(p2w.astype(jnp.bfloat16), wfc_ref[...],
                     preferred_element_type=jnp.float32) + bfc_ref[...]
    o_ref[...] = logits


def _pack_m1_banded(m1):
    zero = jnp.zeros((_W0, _N1), m1.dtype)
    rows = []
    for r in range(8):
        quarters = []
        for q in range(4):
            kh = r - q
            quarters.append(m1[kh] if 0 <= kh < _K else zero)
        rows.append(jnp.concatenate(quarters, axis=1))
    return jnp.concatenate(rows, axis=0)


def _pack_w2_banded(w2p):
    w2r = w2p.reshape(_K, _K, _CP, _CP)
    zero = jnp.zeros((_CP, _CP), w2p.dtype)
    rows = []
    for u in range(3):
        for s in range(2):
            for wpp in range(6):
                cells = []
                for t in range(2):
                    for opp in range(2):
                        kh = 2 * u + s - t
                        kw = wpp - opp
                        ok = 0 <= kh < _K and 0 <= kw < _K
                        cells.append(w2r[kh, kw] if ok else zero)
                rows.append(jnp.concatenate(cells, axis=1))
    return jnp.concatenate(rows, axis=0)


def kernel(x, m1, b1big, w2p, b2p, wfc_flat, bfcp):
    B = x.shape[0]
    xs = x.reshape(B, _H0, _W0).astype(jnp.bfloat16)
    tb = _TB if B >= _TB else B
    Bp = ((B + tb - 1) // tb) * tb
    if Bp != B:
        xs = jnp.pad(xs, ((0, Bp - B), (0, 0), (0, 0)))
    xsplit = [jnp.transpose(xs[:, m::4, :], (1, 0, 2)) for m in range(4)]

    m1b = _pack_m1_banded(m1).astype(jnp.bfloat16)
    b1w = jnp.concatenate([b1big] * 4, axis=1)
    w2b = _pack_w2_banded(w2p).astype(jnp.bfloat16)
    b2w = jnp.concatenate([b2p] * 4, axis=1)
    wfcb = wfc_flat.astype(jnp.bfloat16)

    grid = (Bp // tb,)
    xspec = pl.BlockSpec((7, tb, _W0), lambda i: (0, i, 0))
    out = pl.pallas_call(
        _fused_kernel,
        out_shape=jax.ShapeDtypeStruct((Bp, _CP), jnp.float32),
        grid=grid,
        in_specs=[
            xspec, xspec, xspec, xspec,
            pl.BlockSpec((_KC1, 4 * _N1), lambda i: (0, 0)),
            pl.BlockSpec((1, 4 * _N1), lambda i: (0, 0)),
            pl.BlockSpec((36 * _CP, 4 * _CP), lambda i: (0, 0)),
            pl.BlockSpec((1, 4 * _CP), lambda i: (0, 0)),
            pl.BlockSpec((_PH2 * _PW2 * _CP, _CP), lambda i: (0, 0)),
            pl.BlockSpec((1, _CP), lambda i: (0, 0)),
        ],
        out_specs=pl.BlockSpec((tb, _CP), lambda i: (i, 0)),
        compiler_params=pltpu.CompilerParams(
            dimension_semantics=("parallel",),
            vmem_limit_bytes=64 * 1024 * 1024),
    )(*xsplit, m1b, b1w, w2b, b2w, wfcb, bfcp)

    return out[:B, :10]


# speedup vs baseline: 2.0362x; 1.1326x over previous
import jax
import jax.numpy as jnp
from jax.experimental import pallas as pl
from jax.experimental.pallas import tpu as pltpu

_CP = 128
_K = 5
_H0, _W0 = 28, 28
_OH1, _OW1 = 24, 24
_PH1, _PW1 = 12, 12
_OH2, _OW2 = 8, 8
_PH2, _PW2 = 4, 4
_TB = 64
_N1 = _OW1 * _CP
_KC1 = 8 * _W0
_HALF = _PW1 * _CP


def _fused_kernel(xa_ref, xb_ref, xc_ref, xd_ref, m1_ref, b1_ref, w2_ref,
                  b2_ref, wfc_ref, bfc_ref, o_ref):
    tb = xa_ref.shape[1]
    xq = [xa_ref[...], xb_ref[...], xc_ref[...], xd_ref[...]]

    xk = jnp.concatenate(
        [xq[m][0:6] for m in range(4)] +
        [xq[m][1:7] for m in range(4)], axis=2)
    xk = xk.reshape(6 * tb, _KC1)
    m1w = m1_ref[...]
    hq = []
    for q in range(4):
        aq = jnp.dot(xk[:, q * _W0:q * _W0 + _K * _W0], m1w,
                     preferred_element_type=jnp.float32)
        hq.append(jnp.maximum(aq + b1_ref[...], 0.0))

    def gather(dq, par):
        return jnp.concatenate(
            [hq[2 * s + dq][:, (2 * j + par) * _CP:(2 * j + par + 1) * _CP]
             for s in range(2) for j in range(_PW1)], axis=1)
    p1 = jnp.maximum(jnp.maximum(gather(0, 0), gather(0, 1)),
                     jnp.maximum(gather(1, 0), gather(1, 1)))
    p1 = p1.astype(jnp.bfloat16)

    ljs = []
    for j in range(_PW2):
        ljs.append(jnp.concatenate(
            [p1[u * tb:(u + 4) * tb,
                s * _HALF + 2 * j * _CP:s * _HALF + (2 * j + 6) * _CP]
             for u in range(3) for s in range(2)], axis=1))
    big = jnp.concatenate(ljs, axis=0)
    a0 = jnp.dot(big[:, 0:5 * 768], w2_ref[0],
                 preferred_element_type=jnp.float32)
    a1 = jnp.dot(big[:, 768:6 * 768], w2_ref[1],
                 preferred_element_type=jnp.float32)
    a0 = jnp.maximum(a0 + b2_ref[...], 0.0)
    a1 = jnp.maximum(a1 + b2_ref[...], 0.0)

    pooled = jnp.maximum(
        jnp.maximum(a0[:, 0:_CP], a0[:, _CP:2 * _CP]),
        jnp.maximum(a1[:, 0:_CP], a1[:, _CP:2 * _CP]))

    p2w = jnp.concatenate(
        [pooled[(j * _PH2 + oh2) * tb:(j * _PH2 + oh2 + 1) * tb, :]
         for oh2 in range(_PH2) for j in range(_PW2)], axis=1)
    logits = jnp.dot(p2w.astype(jnp.bfloat16), wfc_ref[...],
                     preferred_element_type=jnp.float32) + bfc_ref[...]
    o_ref[...] = logits


def _pack_m1_banded(m1):
    zero = jnp.zeros((_W0, _N1), m1.dtype)
    rows = []
    for r in range(8):
        quarters = []
        for q in range(4):
            kh = r - q
            quarters.append(m1[kh] if 0 <= kh < _K else zero)
        rows.append(jnp.concatenate(quarters, axis=1))
    return jnp.concatenate(rows, axis=0)


def _pack_w2_banded(w2p):
    w2r = w2p.reshape(_K, _K, _CP, _CP)
    zero = jnp.zeros((_CP, _CP), w2p.dtype)
    rows = []
    for u in range(3):
        for s in range(2):
            for wpp in range(6):
                cells = []
                for t in range(2):
                    for opp in range(2):
                        kh = 2 * u + s - t
                        kw = wpp - opp
                        ok = 0 <= kh < _K and 0 <= kw < _K
                        cells.append(w2r[kh, kw] if ok else zero)
                rows.append(jnp.concatenate(cells, axis=1))
    return jnp.concatenate(rows, axis=0)


def kernel(x, m1, b1big, w2p, b2p, wfc_flat, bfcp):
    B = x.shape[0]
    xs = x.reshape(B, _H0, _W0).astype(jnp.bfloat16)
    tb = _TB if B >= _TB else B
    Bp = ((B + tb - 1) // tb) * tb
    if Bp != B:
        xs = jnp.pad(xs, ((0, Bp - B), (0, 0), (0, 0)))
    xsplit = [jnp.transpose(xs[:, m::4, :], (1, 0, 2)) for m in range(4)]

    m1b = m1.reshape(_K * _W0, _N1).astype(jnp.bfloat16)
    b1w = b1big
    w2full = _pack_w2_banded(w2p).astype(jnp.bfloat16)
    w2b = jnp.stack([w2full[0:5 * 768, 0:256],
                     w2full[768:6 * 768, 256:512]])
    b2w = jnp.concatenate([b2p] * 2, axis=1)
    wfcb = wfc_flat.astype(jnp.bfloat16)

    grid = (Bp // tb,)
    xspec = pl.BlockSpec((7, tb, _W0), lambda i: (0, i, 0))
    out = pl.pallas_call(
        _fused_kernel,
        out_shape=jax.ShapeDtypeStruct((Bp, _CP), jnp.float32),
        grid=grid,
        in_specs=[
            xspec, xspec, xspec, xspec,
            pl.BlockSpec((_K * _W0, _N1), lambda i: (0, 0)),
            pl.BlockSpec((1, _N1), lambda i: (0, 0)),
            pl.BlockSpec((2, 30 * _CP, 2 * _CP), lambda i: (0, 0, 0)),
            pl.BlockSpec((1, 2 * _CP), lambda i: (0, 0)),
            pl.BlockSpec((_PH2 * _PW2 * _CP, _CP), lambda i: (0, 0)),
            pl.BlockSpec((1, _CP), lambda i: (0, 0)),
        ],
        out_specs=pl.BlockSpec((tb, _CP), lambda i: (i, 0)),
        compiler_params=pltpu.CompilerParams(
            dimension_semantics=("parallel",),
            vmem_limit_bytes=64 * 1024 * 1024),
    )(*xsplit, m1b, b1w, w2b, b2w, wfcb, bfcp)

    return out[:B, :10]


# speedup vs baseline: 2.0815x; 1.0222x over previous
import jax
import jax.numpy as jnp
from jax.experimental import pallas as pl
from jax.experimental.pallas import tpu as pltpu

_CP = 128
_K = 5
_H0, _W0 = 28, 28
_OH1, _OW1 = 24, 24
_PH1, _PW1 = 12, 12
_OH2, _OW2 = 8, 8
_PH2, _PW2 = 4, 4
_TB = 128
_N1 = _OW1 * _CP
_KC1 = 8 * _W0
_HALF = _PW1 * _CP


def _fused_kernel(xa_ref, xb_ref, xc_ref, xd_ref, m1_ref, b1_ref, w2_ref,
                  b2_ref, wfc_ref, bfc_ref, o_ref):
    tb = xa_ref.shape[1]
    xq = [xa_ref[...], xb_ref[...], xc_ref[...], xd_ref[...]]

    xk = jnp.concatenate(
        [xq[m][0:6] for m in range(4)] +
        [xq[m][1:7] for m in range(4)], axis=2)
    xk = xk.reshape(6 * tb, _KC1)
    m1w = m1_ref[...]

    def gath(hh, par):
        return jnp.concatenate(
            [hh[:, (2 * j + par) * _CP:(2 * j + par + 1) * _CP]
             for j in range(_PW1)], axis=1)
    halves = []
    for s in range(2):
        hs = []
        for q in (2 * s, 2 * s + 1):
            aq = jnp.dot(xk[:, q * _W0:q * _W0 + _K * _W0], m1w,
                         preferred_element_type=jnp.float32)
            hs.append(jnp.maximum(aq + b1_ref[...], 0.0))
        halves.append(jnp.maximum(
            jnp.maximum(gath(hs[0], 0), gath(hs[0], 1)),
            jnp.maximum(gath(hs[1], 0), gath(hs[1], 1))))
    p1 = jnp.concatenate(halves, axis=1)
    p1 = p1.astype(jnp.bfloat16)

    ljs = []
    for j in range(_PW2):
        ljs.append(jnp.concatenate(
            [p1[u * tb:(u + 4) * tb,
                s * _HALF + 2 * j * _CP:s * _HALF + (2 * j + 6) * _CP]
             for u in range(3) for s in range(2)], axis=1))
    big = jnp.concatenate(ljs, axis=0)
    a0 = jnp.dot(big[:, 0:5 * 768], w2_ref[0],
                 preferred_element_type=jnp.float32)
    a1 = jnp.dot(big[:, 768:6 * 768], w2_ref[1],
                 preferred_element_type=jnp.float32)
    a0 = jnp.maximum(a0 + b2_ref[...], 0.0)
    a1 = jnp.maximum(a1 + b2_ref[...], 0.0)

    pooled = jnp.maximum(
        jnp.maximum(a0[:, 0:_CP], a0[:, _CP:2 * _CP]),
        jnp.maximum(a1[:, 0:_CP], a1[:, _CP:2 * _CP]))

    p2w = jnp.concatenate(
        [pooled[(j * _PH2 + oh2) * tb:(j * _PH2 + oh2 + 1) * tb, :]
         for oh2 in range(_PH2) for j in range(_PW2)], axis=1)
    logits = jnp.dot(p2w.astype(jnp.bfloat16), wfc_ref[...],
                     preferred_element_type=jnp.float32) + bfc_ref[...]
    o_ref[...] = logits


def _pack_m1_banded(m1):
    zero = jnp.zeros((_W0, _N1), m1.dtype)
    rows = []
    for r in range(8):
        quarters = []
        for q in range(4):
            kh = r - q
            quarters.append(m1[kh] if 0 <= kh < _K else zero)
        rows.append(jnp.concatenate(quarters, axis=1))
    return jnp.concatenate(rows, axis=0)


def _pack_w2_banded(w2p):
    w2r = w2p.reshape(_K, _K, _CP, _CP)
    zero = jnp.zeros((_CP, _CP), w2p.dtype)
    rows = []
    for u in range(3):
        for s in range(2):
            for wpp in range(6):
                cells = []
                for t in range(2):
                    for opp in range(2):
                        kh = 2 * u + s - t
                        kw = wpp - opp
                        ok = 0 <= kh < _K and 0 <= kw < _K
                        cells.append(w2r[kh, kw] if ok else zero)
                rows.append(jnp.concatenate(cells, axis=1))
    return jnp.concatenate(rows, axis=0)


def kernel(x, m1, b1big, w2p, b2p, wfc_flat, bfcp):
    B = x.shape[0]
    xs = x.reshape(B, _H0, _W0).astype(jnp.bfloat16)
    tb = _TB if B >= _TB else B
    Bp = ((B + tb - 1) // tb) * tb
    if Bp != B:
        xs = jnp.pad(xs, ((0, Bp - B), (0, 0), (0, 0)))
    xsplit = [jnp.transpose(xs[:, m::4, :], (1, 0, 2)) for m in range(4)]

    m1b = m1.reshape(_K * _W0, _N1).astype(jnp.bfloat16)
    b1w = b1big
    w2full = _pack_w2_banded(w2p).astype(jnp.bfloat16)
    w2b = jnp.stack([w2full[0:5 * 768, 0:256],
                     w2full[768:6 * 768, 256:512]])
    b2w = jnp.concatenate([b2p] * 2, axis=1)
    wfcb = wfc_flat.astype(jnp.bfloat16)

    grid = (Bp // tb,)
    xspec = pl.BlockSpec((7, tb, _W0), lambda i: (0, i, 0))
    out = pl.pallas_call(
        _fused_kernel,
        out_shape=jax.ShapeDtypeStruct((Bp, _CP), jnp.float32),
        grid=grid,
        in_specs=[
            xspec, xspec, xspec, xspec,
            pl.BlockSpec((_K * _W0, _N1), lambda i: (0, 0)),
            pl.BlockSpec((1, _N1), lambda i: (0, 0)),
            pl.BlockSpec((2, 30 * _CP, 2 * _CP), lambda i: (0, 0, 0)),
            pl.BlockSpec((1, 2 * _CP), lambda i: (0, 0)),
            pl.BlockSpec((_PH2 * _PW2 * _CP, _CP), lambda i: (0, 0)),
            pl.BlockSpec((1, _CP), lambda i: (0, 0)),
        ],
        out_specs=pl.BlockSpec((tb, _CP), lambda i: (i, 0)),
        compiler_params=pltpu.CompilerParams(
            dimension_semantics=("parallel",),
            vmem_limit_bytes=64 * 1024 * 1024),
    )(*xsplit, m1b, b1w, w2b, b2w, wfcb, bfcp)

    return out[:B, :10]
